# Initial kernel scaffold; baseline (speedup 1.0000x reference)
import jax
import jax.numpy as jnp
from jax.experimental import pallas as pl


def kernel(inp, target):
    raise NotImplementedError("write your pallas kernel here")


# speedup vs baseline: 1.9386x; 1.9386x over previous
import functools

import jax
import jax.numpy as jnp
from jax import lax
from jax.experimental import pallas as pl
from jax.experimental.pallas import tpu as pltpu
from jax.experimental.pallas import tpu_sc as plsc

N_ROWS = 4096
N_COLS = 32000
NC, NS, L = 2, 16, 16
NW = NC * NS
RPW = N_ROWS // NW
HALF = N_COLS // 2
CHUNKS_H = HALF // L
UNROLL = 8


def _sc_pass(inp_flat, target):
    mesh = plsc.VectorSubcoreMesh(core_axis_name="c", subcore_axis_name="s")

    @functools.partial(
        pl.kernel,
        out_type=(
            jax.ShapeDtypeStruct((N_ROWS * L,), jnp.float32),
            jax.ShapeDtypeStruct((N_ROWS,), jnp.float32),
        ),
        mesh=mesh,
        scratch_types=[
            pltpu.VMEM((2, HALF), jnp.float32),
            pltpu.VMEM((RPW,), jnp.int32),
            pltpu.VMEM((RPW,), jnp.int32),
            pltpu.VMEM((RPW,), jnp.float32),
            pltpu.VMEM((RPW * L,), jnp.float32),
            pltpu.SemaphoreType.DMA,
            pltpu.SemaphoreType.DMA,
            pltpu.SemaphoreType.DMA,
        ],
    )
    def k(inp_hbm, tgt_hbm, s_out, x_out,
          buf, tgt_v, idx_v, xt_v, s_stage, sem0, sem1, semg):
        wid = lax.axis_index("s") * NC + lax.axis_index("c")
        base = wid * RPW

        pltpu.sync_copy(tgt_hbm.at[pl.ds(base, RPW)], tgt_v)

        def mk_idx(g, carry):
            tv = tgt_v[pl.ds(g * L, L)]
            rows = (base + g * L) + lax.iota(jnp.int32, L)
            idx_v[pl.ds(g * L, L)] = rows * N_COLS + tv
            return carry

        lax.fori_loop(0, RPW // L, mk_idx, 0)
        gather = pltpu.async_copy(inp_hbm.at[idx_v], xt_v, semg)

        sems = (sem0, sem1)
        for h in range(2):
            off0 = base * N_COLS + h * HALF
            pltpu.async_copy(inp_hbm.at[pl.ds(off0, HALF)], buf.at[h], sems[h])

        def row_body(j, carry):
            acc = jnp.zeros((L,), jnp.float32)
            for h in range(2):
                pltpu.make_async_copy(
                    inp_hbm.at[pl.ds(0, HALF)], buf.at[h], sems[h]).wait()
                bh = buf.at[h]

                def chunk(c, a):
                    o = c * (L * UNROLL)
                    for u in range(UNROLL):
                        a = a + jnp.exp(bh[pl.ds(o + u * L, L)])
                    return a

                acc = lax.fori_loop(0, CHUNKS_H // UNROLL, chunk, acc)

                @pl.when(j + 1 < RPW)
                def _():
                    off = (base + j + 1) * N_COLS + h * HALF
                    pltpu.async_copy(
                        inp_hbm.at[pl.ds(off, HALF)], buf.at[h], sems[h])

            s_stage[pl.ds(j * L, L)] = acc
            return carry

        lax.fori_loop(0, RPW, row_body, 0)

        gather.wait()
        pltpu.sync_copy(s_stage, s_out.at[pl.ds(base * L, RPW * L)])
        pltpu.sync_copy(xt_v, x_out.at[pl.ds(base, RPW)])

    return k(inp_flat, target)


def _finish(s2, xt, tgt):
    def fk(s_ref, x_ref, t_ref, o_ref):
        s_sum = jnp.sum(s_ref[...], axis=1, keepdims=True)
        mask = t_ref[...] != 0
        loss = jnp.sum(jnp.where(mask, jnp.log(s_sum) - x_ref[...], 0.0))
        o_ref[...] = jnp.full((1, 1), loss, jnp.float32)

    return pl.pallas_call(
        fk, out_shape=jax.ShapeDtypeStruct((1, 1), jnp.float32))(s2, xt, tgt)


def kernel(inp, target):
    inp_flat = inp.reshape(-1)
    tgt = target.astype(jnp.int32)
    s_out, x_out = _sc_pass(inp_flat, tgt)
    out = _finish(s_out.reshape(N_ROWS, L),
                  x_out.reshape(N_ROWS, 1),
                  tgt.reshape(N_ROWS, 1))
    return out[0, 0]


# speedup vs baseline: 2.2492x; 1.1602x over previous
import functools

import jax
import jax.numpy as jnp
from jax import lax
from jax.experimental import pallas as pl
from jax.experimental.pallas import tpu as pltpu
from jax.experimental.pallas import tpu_sc as plsc

N_ROWS = 4096
N_COLS = 32000
NC, NS, L = 2, 16, 16
NW = NC * NS
RPW = N_ROWS // NW
HALF = N_COLS // 2
CHUNKS_H = HALF // L
UNROLL = 8


def _sc_pass(inp_flat, target):
    mesh = plsc.VectorSubcoreMesh(core_axis_name="c", subcore_axis_name="s")

    @functools.partial(
        pl.kernel,
        out_type=(
            jax.ShapeDtypeStruct((N_ROWS * L,), jnp.float32),
            jax.ShapeDtypeStruct((N_ROWS,), jnp.float32),
        ),
        mesh=mesh,
        scratch_types=[
            pltpu.VMEM((2, HALF), jnp.float32),
            pltpu.VMEM((RPW,), jnp.int32),
            pltpu.VMEM((RPW,), jnp.int32),
            pltpu.VMEM((RPW,), jnp.float32),
            pltpu.VMEM((RPW * L,), jnp.float32),
            pltpu.SemaphoreType.DMA,
            pltpu.SemaphoreType.DMA,
            pltpu.SemaphoreType.DMA,
        ],
    )
    def k(inp_hbm, tgt_hbm, s_out, x_out,
          buf, tgt_v, idx_v, xt_v, s_stage, sem0, sem1, semg):
        wid = lax.axis_index("s") * NC + lax.axis_index("c")
        base = wid * RPW

        pltpu.sync_copy(tgt_hbm.at[pl.ds(base, RPW)], tgt_v)

        def mk_idx(g, carry):
            tv = tgt_v[pl.ds(g * L, L)]
            rows = (base + g * L) + lax.iota(jnp.int32, L)
            idx_v[pl.ds(g * L, L)] = rows * N_COLS + tv
            return carry

        lax.fori_loop(0, RPW // L, mk_idx, 0)
        gather = pltpu.async_copy(inp_hbm.at[idx_v], xt_v, semg)

        sems = (sem0, sem1)
        for h in range(2):
            off0 = base * N_COLS + h * HALF
            pltpu.async_copy(inp_hbm.at[pl.ds(off0, HALF)], buf.at[h], sems[h])

        def row_body(j, carry):
            zero = jnp.zeros((L,), jnp.float32)
            accs = (zero, zero, zero, zero)
            for h in range(2):
                pltpu.make_async_copy(
                    inp_hbm.at[pl.ds(0, HALF)], buf.at[h], sems[h]).wait()
                bh = buf.at[h]

                @plsc.parallel_loop(0, CHUNKS_H, step=UNROLL, unroll=2,
                                    carry=accs)
                def accs(c, accs):
                    a0, a1, a2, a3 = accs
                    o = c * L
                    a0 = a0 + jnp.exp(bh[pl.ds(o + 0 * L, L)])
                    a1 = a1 + jnp.exp(bh[pl.ds(o + 1 * L, L)])
                    a2 = a2 + jnp.exp(bh[pl.ds(o + 2 * L, L)])
                    a3 = a3 + jnp.exp(bh[pl.ds(o + 3 * L, L)])
                    a0 = a0 + jnp.exp(bh[pl.ds(o + 4 * L, L)])
                    a1 = a1 + jnp.exp(bh[pl.ds(o + 5 * L, L)])
                    a2 = a2 + jnp.exp(bh[pl.ds(o + 6 * L, L)])
                    a3 = a3 + jnp.exp(bh[pl.ds(o + 7 * L, L)])
                    return (a0, a1, a2, a3)

                @pl.when(j + 1 < RPW)
                def _():
                    off = (base + j + 1) * N_COLS + h * HALF
                    pltpu.async_copy(
                        inp_hbm.at[pl.ds(off, HALF)], buf.at[h], sems[h])

            a0, a1, a2, a3 = accs
            s_stage[pl.ds(j * L, L)] = (a0 + a1) + (a2 + a3)
            return carry

        lax.fori_loop(0, RPW, row_body, 0)

        gather.wait()
        pltpu.sync_copy(s_stage, s_out.at[pl.ds(base * L, RPW * L)])
        pltpu.sync_copy(xt_v, x_out.at[pl.ds(base, RPW)])

    return k(inp_flat, target)


def _finish(s2, xt, tgt):
    def fk(s_ref, x_ref, t_ref, o_ref):
        s_sum = jnp.sum(s_ref[...], axis=1, keepdims=True)
        mask = t_ref[...] != 0
        loss = jnp.sum(jnp.where(mask, jnp.log(s_sum) - x_ref[...], 0.0))
        o_ref[...] = jnp.full((1, 1), loss, jnp.float32)

    return pl.pallas_call(
        fk, out_shape=jax.ShapeDtypeStruct((1, 1), jnp.float32))(s2, xt, tgt)


def kernel(inp, target):
    inp_flat = inp.reshape(-1)
    tgt = target.astype(jnp.int32)
    s_out, x_out = _sc_pass(inp_flat, tgt)
    out = _finish(s_out.reshape(N_ROWS, L),
                  x_out.reshape(N_ROWS, 1),
                  tgt.reshape(N_ROWS, 1))
    return out[0, 0]


# speedup vs baseline: 5.7505x; 2.5567x over previous
import functools

import jax
import jax.numpy as jnp
from jax import lax
from jax.experimental import pallas as pl
from jax.experimental.pallas import tpu as pltpu
from jax.experimental.pallas import tpu_sc as plsc

N_ROWS = 4096
N_COLS = 32000
NC, NS, L = 2, 16, 16
NW = NC * NS
RPW = N_ROWS // NW
HALF = N_COLS // 2
CHUNKS_H = HALF // L
UNROLL = 8


def _sc_pass(inp, target):
    mesh = plsc.VectorSubcoreMesh(core_axis_name="c", subcore_axis_name="s")

    @functools.partial(
        pl.kernel,
        out_type=(
            jax.ShapeDtypeStruct((N_ROWS * L,), jnp.float32),
            jax.ShapeDtypeStruct((N_ROWS * L,), jnp.float32),
        ),
        mesh=mesh,
        compiler_params=pltpu.CompilerParams(needs_layout_passes=False),
        scratch_types=[
            pltpu.VMEM((HALF,), jnp.float32),
            pltpu.VMEM((HALF,), jnp.float32),
            pltpu.VMEM((RPW,), jnp.int32),
            pltpu.VMEM((RPW * L,), jnp.float32),
            pltpu.VMEM((RPW * L,), jnp.float32),
            pltpu.SemaphoreType.DMA,
            pltpu.SemaphoreType.DMA,
        ],
    )
    def k(inp_hbm, tgt_hbm, s_out, x_out,
          buf0, buf1, tgt_v, s_stage, x_stage, sem0, sem1):
        bufs = (buf0, buf1)
        wid = lax.axis_index("s") * NC + lax.axis_index("c")
        base = wid * RPW

        pltpu.sync_copy(tgt_hbm.at[pl.ds(base, RPW)], tgt_v)

        sems = (sem0, sem1)
        for h in range(2):
            pltpu.async_copy(
                inp_hbm.at[base, pl.ds(h * HALF, HALF)], bufs[h], sems[h])

        def row_body(j, carry):
            zero = jnp.zeros((L,), jnp.float32)
            accs = (zero, zero, zero, zero)
            t_vec = tgt_v[pl.ds((j // L) * L, L)]
            row_hot = lax.iota(jnp.int32, L) == (j % L)
            t = jnp.max(jnp.where(row_hot, t_vec, 0))
            for h in range(2):
                pltpu.make_async_copy(
                    inp_hbm.at[0, pl.ds(0, HALF)], bufs[h], sems[h]).wait()
                bh = bufs[h]

                @plsc.parallel_loop(0, CHUNKS_H, step=UNROLL, unroll=2,
                                    carry=accs)
                def accs(c, accs):
                    a0, a1, a2, a3 = accs
                    o = c * L
                    a0 = a0 + jnp.exp(bh[pl.ds(o + 0 * L, L)])
                    a1 = a1 + jnp.exp(bh[pl.ds(o + 1 * L, L)])
                    a2 = a2 + jnp.exp(bh[pl.ds(o + 2 * L, L)])
                    a3 = a3 + jnp.exp(bh[pl.ds(o + 3 * L, L)])
                    a0 = a0 + jnp.exp(bh[pl.ds(o + 4 * L, L)])
                    a1 = a1 + jnp.exp(bh[pl.ds(o + 5 * L, L)])
                    a2 = a2 + jnp.exp(bh[pl.ds(o + 6 * L, L)])
                    a3 = a3 + jnp.exp(bh[pl.ds(o + 7 * L, L)])
                    return (a0, a1, a2, a3)

                local = t - h * HALF

                @pl.when((local >= 0) & (local < HALF))
                def _():
                    c0 = (local // L) * L
                    chunkv = bh[pl.ds(c0, L)]
                    onehot = lax.iota(jnp.int32, L) == (local - c0)
                    x_stage[pl.ds(j * L, L)] = jnp.where(onehot, chunkv, 0.0)

                @pl.when(j + 1 < RPW)
                def _():
                    pltpu.async_copy(
                        inp_hbm.at[base + j + 1, pl.ds(h * HALF, HALF)],
                        bufs[h], sems[h])

            a0, a1, a2, a3 = accs
            s_stage[pl.ds(j * L, L)] = (a0 + a1) + (a2 + a3)
            return carry

        lax.fori_loop(0, RPW, row_body, 0)

        pltpu.sync_copy(s_stage, s_out.at[pl.ds(base * L, RPW * L)])
        pltpu.sync_copy(x_stage, x_out.at[pl.ds(base * L, RPW * L)])

    return k(inp, target)


def _finish(s2, x2, tgt):
    def fk(s_ref, x_ref, t_ref, o_ref):
        s_sum = jnp.sum(s_ref[...], axis=1, keepdims=True)
        mask = t_ref[...] != 0
        xt = jnp.sum(x_ref[...], axis=1, keepdims=True)
        loss = jnp.sum(jnp.where(mask, jnp.log(s_sum) - xt, 0.0))
        o_ref[...] = jnp.full((1, 1), loss, jnp.float32)

    return pl.pallas_call(
        fk, out_shape=jax.ShapeDtypeStruct((1, 1), jnp.float32))(s2, x2, tgt)


def kernel(inp, target):
    tgt = target.astype(jnp.int32)
    s_out, x_out = _sc_pass(inp, tgt)
    out = _finish(s_out.reshape(N_ROWS, L),
                  x_out.reshape(N_ROWS, L),
                  tgt.reshape(N_ROWS, 1))
    return out[0, 0]
